# baseline (speedup 1.0000x reference)
import functools

import jax
import jax.numpy as jnp
from jax import lax
from jax.experimental import pallas as pl
from jax.experimental.pallas import tpu as pltpu
from jax.experimental.pallas import tpu_sc as plsc

_DIM = 32
_BT = 128
_NG = 4
_NS = 2


def _fuse_body(tab_ref, pos_ref, out_ref):
    out_ref[...] = tab_ref[...] + pos_ref[...]


def _fuse_tables(table, pos_enc):
    n, d = table.shape
    rows = (n * d) // 128
    blk = 1000
    t2 = table.reshape(rows, 128)
    p2 = pos_enc.reshape(rows, 128)
    fused = pl.pallas_call(
        _fuse_body,
        grid=(rows // blk,),
        in_specs=[
            pl.BlockSpec((blk, 128), lambda i: (i, 0)),
            pl.BlockSpec((blk, 128), lambda i: (i, 0)),
        ],
        out_specs=pl.BlockSpec((blk, 128), lambda i: (i, 0)),
        out_shape=jax.ShapeDtypeStruct((rows, 128), table.dtype),
    )(t2, p2)
    return fused.reshape(n, d)


@functools.lru_cache(maxsize=None)
def _make_gather(batch, hist):
    info = plsc.get_sparse_core_info()
    nc, ns = info.num_cores, info.num_subcores
    nw = nc * ns
    n_bt = batch // _BT
    bt_per_w = n_bt // nw
    n_th = hist // 8
    n_td = _DIM // 8
    assert batch % (_BT * nw) == 0 and hist % 8 == 0
    mesh = plsc.VectorSubcoreMesh(core_axis_name="c", subcore_axis_name="s")

    @functools.partial(
        pl.kernel,
        mesh=mesh,
        compiler_params=pltpu.CompilerParams(
            use_tc_tiling_on_sc=False, needs_layout_passes=False),
        out_type=jax.ShapeDtypeStruct((hist, n_td, n_bt, 8, _BT), jnp.float32),
        scratch_types=[
            pltpu.VMEM((n_th, 8, _BT), jnp.int32),
        ]
        + [pltpu.VMEM((_BT, _DIM), jnp.float32) for _ in range(_NG)]
        + [pltpu.VMEM((n_td, 8, _BT), jnp.float32) for _ in range(_NS)]
        + [pltpu.SemaphoreType.DMA for _ in range(_NG + _NS)],
    )
    def gather(x4_hbm, tab_hbm, out_hbm, idx_v, *bufs):
        rows = bufs[:_NG]
        trs = bufs[_NG:_NG + _NS]
        gsems = bufs[_NG + _NS:2 * _NG + _NS]
        ssems = bufs[2 * _NG + _NS:]
        wid = lax.axis_index("s") * nc + lax.axis_index("c")
        lanes = jnp.arange(16, dtype=jnp.int32)

        def transpose_cell(rows_v, tr_v):
            @plsc.parallel_loop(0, _BT // 16, unroll=4)
            def grp(g):
                row_ids = g * 16 + lanes
                for d in range(_DIM):
                    col_ids = jnp.full((16,), d, jnp.int32)
                    vals = plsc.load_gather(rows_v, [row_ids, col_ids])
                    tr_v[d // 8, d % 8, pl.ds(g * 16, 16)] = vals

        def fire_gather(h, rows_v, sem):
            return pltpu.async_copy(
                tab_hbm.at[idx_v.at[h // 8, h % 8]], rows_v, sem)

        for k in range(bt_per_w):
            tb = wid * bt_per_w + k
            pltpu.sync_copy(x4_hbm.at[:, tb], idx_v)
            for j in range(_NG):
                fire_gather(j, rows[j], gsems[j])

            def group(g, carry):
                for j in range(_NG):
                    cell = g * _NG + j
                    tj = j % _NS
                    pltpu.make_async_copy(tab_hbm.at[idx_v.at[0, 0]],
                                          rows[j], gsems[j]).wait()
                    if j >= _NS:
                        pltpu.make_async_copy(trs[tj], out_hbm.at[0, :, tb],
                                              ssems[tj]).wait()
                    else:
                        @pl.when(g > 0)
                        def _():
                            pltpu.make_async_copy(
                                trs[tj], out_hbm.at[0, :, tb],
                                ssems[tj]).wait()

                    transpose_cell(rows[j], trs[tj])
                    pltpu.async_copy(trs[tj], out_hbm.at[cell, :, tb],
                                     ssems[tj])

                    @pl.when(cell + _NG < hist)
                    def _():
                        fire_gather(cell + _NG, rows[j], gsems[j])
                return carry

            lax.fori_loop(0, hist // _NG, group, 0)
            for tj in range(_NS):
                pltpu.make_async_copy(trs[tj], out_hbm.at[0, :, tb],
                                      ssems[tj]).wait()

    return gather


def kernel(x, table, pos_enc):
    b, h = x.shape
    fused = _fuse_tables(table, pos_enc)
    x4 = (
        x.astype(jnp.int32)
        .T.reshape(h // 8, 8, b // _BT, _BT)
        .transpose(0, 2, 1, 3)
    )
    out5 = _make_gather(b, h)(x4, fused)
    return jnp.transpose(out5, (2, 4, 0, 1, 3)).reshape(b, h, _DIM)


# speedup vs baseline: 4.3002x; 4.3002x over previous
import functools

import jax
import jax.numpy as jnp
from jax import lax
from jax.experimental import pallas as pl
from jax.experimental.pallas import tpu as pltpu
from jax.experimental.pallas import tpu_sc as plsc

_DIM = 32
_BT = 128
_NG = 4
_NS = 2


def _fuse_body(tab_ref, pos_ref, out_ref):
    out_ref[...] = tab_ref[...] + pos_ref[...]


def _fuse_tables(table, pos_enc):
    n, d = table.shape
    rows = (n * d) // 128
    blk = 1000
    t2 = table.reshape(rows, 128)
    p2 = pos_enc.reshape(rows, 128)
    fused = pl.pallas_call(
        _fuse_body,
        grid=(rows // blk,),
        in_specs=[
            pl.BlockSpec((blk, 128), lambda i: (i, 0)),
            pl.BlockSpec((blk, 128), lambda i: (i, 0)),
        ],
        out_specs=pl.BlockSpec((blk, 128), lambda i: (i, 0)),
        out_shape=jax.ShapeDtypeStruct((rows, 128), table.dtype),
    )(t2, p2)
    return fused.reshape(n, d)


@functools.lru_cache(maxsize=None)
def _make_gather(batch, hist):
    info = plsc.get_sparse_core_info()
    nc, ns = info.num_cores, info.num_subcores
    nw = nc * ns
    n_bt = batch // _BT
    bt_per_w = n_bt // nw
    n_th = hist // 8
    n_td = _DIM // 8
    assert batch % (_BT * nw) == 0 and hist % 8 == 0
    mesh = plsc.VectorSubcoreMesh(core_axis_name="c", subcore_axis_name="s")

    @functools.partial(
        pl.kernel,
        mesh=mesh,
        compiler_params=pltpu.CompilerParams(
            use_tc_tiling_on_sc=False, needs_layout_passes=False),
        out_type=jax.ShapeDtypeStruct((hist, n_td, n_bt, 8, _BT), jnp.float32),
        scratch_types=[
            pltpu.VMEM((n_th, 8, _BT), jnp.int32),
        ]
        + [pltpu.VMEM((_BT, _DIM), jnp.float32) for _ in range(_NG)]
        + [pltpu.VMEM((n_td, 8, _BT + 1), jnp.float32) for _ in range(_NS)]
        + [pltpu.SemaphoreType.DMA for _ in range(_NG + _NS)],
    )
    def gather(x4_hbm, tab_hbm, out_hbm, idx_v, *bufs):
        rows = bufs[:_NG]
        trs = bufs[_NG:_NG + _NS]
        gsems = bufs[_NG + _NS:2 * _NG + _NS]
        ssems = bufs[2 * _NG + _NS:]
        wid = lax.axis_index("s") * nc + lax.axis_index("c")
        lanes = jnp.arange(16, dtype=jnp.int32)

        def transpose_cell(rows_v, tr_v):
            @plsc.parallel_loop(0, _BT, unroll=4)
            def per_b(b):
                b_ids = jnp.full((16,), b, jnp.int32)
                for d0 in (0, 16):
                    vals = rows_v[b, pl.ds(d0, 16)]
                    d_ids = d0 + lanes
                    plsc.store_scatter(
                        tr_v, [d_ids >> 3, d_ids & 7, b_ids], vals)

        def fire_gather(h, rows_v, sem):
            return pltpu.async_copy(
                tab_hbm.at[idx_v.at[h // 8, h % 8]], rows_v, sem)

        for k in range(bt_per_w):
            tb = wid * bt_per_w + k
            pltpu.sync_copy(x4_hbm.at[:, tb], idx_v)
            for j in range(_NG):
                fire_gather(j, rows[j], gsems[j])

            def group(g, carry):
                for j in range(_NG):
                    cell = g * _NG + j
                    tj = j % _NS
                    pltpu.make_async_copy(tab_hbm.at[idx_v.at[0, 0]],
                                          rows[j], gsems[j]).wait()
                    if j >= _NS:
                        pltpu.make_async_copy(trs[tj].at[:, :, pl.ds(0, _BT)],
                                              out_hbm.at[0, :, tb],
                                              ssems[tj]).wait()
                    else:
                        @pl.when(g > 0)
                        def _():
                            pltpu.make_async_copy(
                                trs[tj].at[:, :, pl.ds(0, _BT)],
                                out_hbm.at[0, :, tb], ssems[tj]).wait()

                    transpose_cell(rows[j], trs[tj])
                    pltpu.async_copy(trs[tj].at[:, :, pl.ds(0, _BT)],
                                     out_hbm.at[cell, :, tb],
                                     ssems[tj])

                    @pl.when(cell + _NG < hist)
                    def _():
                        fire_gather(cell + _NG, rows[j], gsems[j])
                return carry

            lax.fori_loop(0, hist // _NG, group, 0)
            for tj in range(_NS):
                pltpu.make_async_copy(trs[tj].at[:, :, pl.ds(0, _BT)],
                                              out_hbm.at[0, :, tb],
                                      ssems[tj]).wait()

    return gather


def kernel(x, table, pos_enc):
    b, h = x.shape
    fused = _fuse_tables(table, pos_enc)
    x4 = (
        x.astype(jnp.int32)
        .T.reshape(h // 8, 8, b // _BT, _BT)
        .transpose(0, 2, 1, 3)
    )
    out5 = _make_gather(b, h)(x4, fused)
    return jnp.transpose(out5, (2, 4, 0, 1, 3)).reshape(b, h, _DIM)


# speedup vs baseline: 4.5285x; 1.0531x over previous
import functools

import jax
import jax.numpy as jnp
from jax import lax
from jax.experimental import pallas as pl
from jax.experimental.pallas import tpu as pltpu
from jax.experimental.pallas import tpu_sc as plsc

_DIM = 32
_BT = 128
_NG = 8
_NS = 4


def _fuse_body(tab_ref, pos_ref, out_ref):
    out_ref[...] = tab_ref[...] + pos_ref[...]


def _fuse_tables(table, pos_enc):
    n, d = table.shape
    rows = (n * d) // 128
    blk = 1000
    t2 = table.reshape(rows, 128)
    p2 = pos_enc.reshape(rows, 128)
    fused = pl.pallas_call(
        _fuse_body,
        grid=(rows // blk,),
        in_specs=[
            pl.BlockSpec((blk, 128), lambda i: (i, 0)),
            pl.BlockSpec((blk, 128), lambda i: (i, 0)),
        ],
        out_specs=pl.BlockSpec((blk, 128), lambda i: (i, 0)),
        out_shape=jax.ShapeDtypeStruct((rows, 128), table.dtype),
    )(t2, p2)
    return fused.reshape(n, d)


@functools.lru_cache(maxsize=None)
def _make_gather(batch, hist):
    info = plsc.get_sparse_core_info()
    nc, ns = info.num_cores, info.num_subcores
    nw = nc * ns
    n_bt = batch // _BT
    bt_per_w = n_bt // nw
    n_th = hist // 8
    n_td = _DIM // 8
    assert batch % (_BT * nw) == 0 and hist % 8 == 0
    mesh = plsc.VectorSubcoreMesh(core_axis_name="c", subcore_axis_name="s")

    @functools.partial(
        pl.kernel,
        mesh=mesh,
        compiler_params=pltpu.CompilerParams(
            use_tc_tiling_on_sc=False, needs_layout_passes=False),
        out_type=jax.ShapeDtypeStruct((hist, n_td, n_bt, 8, _BT), jnp.float32),
        scratch_types=[
            pltpu.VMEM((n_th, 8, _BT), jnp.int32),
        ]
        + [pltpu.VMEM((_BT, _DIM), jnp.float32) for _ in range(_NG)]
        + [pltpu.VMEM((n_td, 8, _BT + 1), jnp.float32) for _ in range(_NS)]
        + [pltpu.SemaphoreType.DMA for _ in range(_NG + _NS)],
    )
    def gather(x4_hbm, tab_hbm, out_hbm, idx_v, *bufs):
        rows = bufs[:_NG]
        trs = bufs[_NG:_NG + _NS]
        gsems = bufs[_NG + _NS:2 * _NG + _NS]
        ssems = bufs[2 * _NG + _NS:]
        wid = lax.axis_index("s") * nc + lax.axis_index("c")
        lanes = jnp.arange(16, dtype=jnp.int32)

        def transpose_cell(rows_v, tr_v):
            @plsc.parallel_loop(0, _BT, unroll=4)
            def per_b(b):
                b_ids = jnp.full((16,), b, jnp.int32)
                for d0 in (0, 16):
                    vals = rows_v[b, pl.ds(d0, 16)]
                    d_ids = d0 + lanes
                    plsc.store_scatter(
                        tr_v, [d_ids >> 3, d_ids & 7, b_ids], vals)

        def fire_gather(h, rows_v, sem):
            return pltpu.async_copy(
                tab_hbm.at[idx_v.at[h // 8, h % 8]], rows_v, sem)

        for k in range(bt_per_w):
            tb = wid * bt_per_w + k
            pltpu.sync_copy(x4_hbm.at[:, tb], idx_v)
            for j in range(_NG):
                fire_gather(j, rows[j], gsems[j])

            def group(g, carry):
                for j in range(_NG):
                    cell = g * _NG + j
                    tj = j % _NS
                    pltpu.make_async_copy(tab_hbm.at[idx_v.at[0, 0]],
                                          rows[j], gsems[j]).wait()
                    if j >= _NS:
                        pltpu.make_async_copy(trs[tj].at[:, :, pl.ds(0, _BT)],
                                              out_hbm.at[0, :, tb],
                                              ssems[tj]).wait()
                    else:
                        @pl.when(g > 0)
                        def _():
                            pltpu.make_async_copy(
                                trs[tj].at[:, :, pl.ds(0, _BT)],
                                out_hbm.at[0, :, tb], ssems[tj]).wait()

                    transpose_cell(rows[j], trs[tj])
                    pltpu.async_copy(trs[tj].at[:, :, pl.ds(0, _BT)],
                                     out_hbm.at[cell, :, tb],
                                     ssems[tj])

                    @pl.when(cell + _NG < hist)
                    def _():
                        fire_gather(cell + _NG, rows[j], gsems[j])
                return carry

            lax.fori_loop(0, hist // _NG, group, 0)
            for tj in range(_NS):
                pltpu.make_async_copy(trs[tj].at[:, :, pl.ds(0, _BT)],
                                              out_hbm.at[0, :, tb],
                                      ssems[tj]).wait()

    return gather


def kernel(x, table, pos_enc):
    b, h = x.shape
    fused = _fuse_tables(table, pos_enc)
    x4 = (
        x.astype(jnp.int32)
        .T.reshape(h // 8, 8, b // _BT, _BT)
        .transpose(0, 2, 1, 3)
    )
    out5 = _make_gather(b, h)(x4, fused)
    return jnp.transpose(out5, (2, 4, 0, 1, 3)).reshape(b, h, _DIM)
